# baseline (speedup 1.0000x reference)
import functools

import jax
import jax.numpy as jnp
from jax import lax
from jax.experimental import pallas as pl
from jax.experimental.pallas import tpu as pltpu
from jax.experimental.pallas import tpu_sc as plsc

_NC, _NS = 2, 16
_NW = _NC * _NS
_S = 8
_NSLOT = 3


def _sc_posenc(B, T, D):
    pos_per_w = T // _NW
    n_chunks = pos_per_w // _S
    n_groups = n_chunks // _NSLOT
    tail = n_chunks - n_groups * _NSLOT
    mesh = plsc.VectorSubcoreMesh(core_axis_name="c", subcore_axis_name="s")

    @functools.partial(
        pl.kernel,
        out_type=jax.ShapeDtypeStruct((B * T, D), jnp.float32),
        mesh=mesh,
        scratch_types=[
            pltpu.VMEM((_NSLOT, _S, D), jnp.float32),
            pltpu.VMEM((_NSLOT, 4, _S, D), jnp.float32),
            pltpu.SemaphoreType.DMA,
            pltpu.SemaphoreType.DMA,
            pltpu.SemaphoreType.DMA,
            pltpu.SemaphoreType.DMA,
            pltpu.SemaphoreType.DMA,
            pltpu.SemaphoreType.DMA,
            pltpu.SemaphoreType.DMA,
            pltpu.SemaphoreType.DMA,
            pltpu.SemaphoreType.DMA,
        ],
    )
    def body(x_hbm, emb_hbm, out_hbm, ebuf, xbuf,
             es0, es1, es2, xs0, xs1, xs2, os0, os1, os2):
        esems = (es0, es1, es2)
        xsems = (xs0, xs1, xs2)
        osems = (os0, os1, os2)
        wid = lax.axis_index("s") * _NC + lax.axis_index("c")
        p0 = wid * pos_per_w

        def eload(g, s):
            return pltpu.make_async_copy(
                emb_hbm.at[pl.ds(p0 + g * _S, _S)], ebuf.at[s], esems[s])

        def xload(g, s, b):
            return pltpu.make_async_copy(
                x_hbm.at[pl.ds(b * T + p0 + g * _S, _S)], xbuf.at[s, b],
                xsems[s])

        def ostore(g, s, b):
            return pltpu.make_async_copy(
                xbuf.at[s, b], out_hbm.at[pl.ds(b * T + p0 + g * _S, _S)],
                osems[s])

        def start_chunk(g, s):
            eload(g, s).start()
            for b in range(4):
                xload(g, s, b).start()

        def chunk_work(g, s):
            eload(g, s).wait()
            for b in range(4):
                xload(g, s, b).wait()

            @plsc.parallel_loop(0, _S, 1, unroll=4)
            def add_body(r):
                for j in range(D // 16):
                    v = ebuf[s, r, pl.ds(j * 16, 16)]
                    for b in range(4):
                        plsc.addupdate(
                            xbuf.at[s, b, r, pl.ds(j * 16, 16)], v)

            for b in range(4):
                ostore(g, s, b).start()

            @pl.when(g + 2 <= n_chunks - 1)
            def _():
                @pl.when(g >= 1)
                def _():
                    for b in range(4):
                        ostore(g - 1, (s + 2) % _NSLOT, b).wait()

                start_chunk(g + 2, (s + 2) % _NSLOT)

        start_chunk(0, 0)
        start_chunk(1, 1)

        def group(gg, carry):
            for p in range(_NSLOT):
                chunk_work(gg * _NSLOT + p, p)
            return carry

        lax.fori_loop(0, n_groups, group, 0)
        for i in range(tail):
            chunk_work(n_groups * _NSLOT + i, i)

        for g in range(n_chunks - _NSLOT, n_chunks):
            for b in range(4):
                ostore(g, g % _NSLOT, b).wait()

    return body


def kernel(x, emb):
    B, T, D = x.shape
    assert T % (_NW * _S) == 0 and D % 16 == 0
    out = _sc_posenc(B, T, D)(x.reshape(B * T, D), emb)
    return out.reshape(B, T, D)


# speedup vs baseline: 1.4210x; 1.4210x over previous
import functools

import jax
import jax.numpy as jnp
from jax import lax
from jax.experimental import pallas as pl
from jax.experimental.pallas import tpu as pltpu
from jax.experimental.pallas import tpu_sc as plsc

_NC, _NS = 2, 16
_NW = _NC * _NS
_S = 8
_NSLOT = 3


def _sc_posenc(B, T, D):
    pos_per_w = T // _NW
    n_chunks = pos_per_w // _S
    n_groups = n_chunks // _NSLOT
    tail = n_chunks - n_groups * _NSLOT
    mesh = plsc.VectorSubcoreMesh(core_axis_name="c", subcore_axis_name="s")

    @functools.partial(
        pl.kernel,
        out_type=jax.ShapeDtypeStruct((B * T, D), jnp.float32),
        mesh=mesh,
        scratch_types=[
            pltpu.VMEM((_NSLOT, _S, D), jnp.float32),
            pltpu.VMEM((_NSLOT, 4, _S, D), jnp.float32),
            pltpu.SemaphoreType.DMA,
            pltpu.SemaphoreType.DMA,
            pltpu.SemaphoreType.DMA,
            pltpu.SemaphoreType.DMA,
            pltpu.SemaphoreType.DMA,
            pltpu.SemaphoreType.DMA,
            pltpu.SemaphoreType.DMA,
            pltpu.SemaphoreType.DMA,
            pltpu.SemaphoreType.DMA,
        ],
    )
    def body(x_hbm, emb_hbm, out_hbm, ebuf, xbuf,
             es0, es1, es2, xs0, xs1, xs2, os0, os1, os2):
        esems = (es0, es1, es2)
        xsems = (xs0, xs1, xs2)
        osems = (os0, os1, os2)
        wid = lax.axis_index("s") * _NC + lax.axis_index("c")
        p0 = wid * pos_per_w

        def eload(g, s):
            return pltpu.make_async_copy(
                emb_hbm.at[pl.ds(p0 + g * _S, _S)], ebuf.at[s], esems[s])

        def xload(g, s, b):
            return pltpu.make_async_copy(
                x_hbm.at[pl.ds(b * T + p0 + g * _S, _S)], xbuf.at[s, b],
                xsems[s])

        def ostore(g, s, b):
            return pltpu.make_async_copy(
                xbuf.at[s, b], out_hbm.at[pl.ds(b * T + p0 + g * _S, _S)],
                osems[s])

        def start_chunk(g, s):
            eload(g, s).start()
            for b in range(4):
                xload(g, s, b).start()

        def chunk_work(g, s):
            eload(g, s).wait()
            for b in range(4):
                xload(g, s, b).wait()

            @plsc.parallel_loop(0, _S, 1, unroll=1)
            def add_body(r):
                for j in range(D // 16):
                    v = ebuf[s, r, pl.ds(j * 16, 16)]
                    for b in range(4):
                        plsc.addupdate(
                            xbuf.at[s, b, r, pl.ds(j * 16, 16)], v)

            for b in range(4):
                ostore(g, s, b).start()

            @pl.when(g + 2 <= n_chunks - 1)
            def _():
                @pl.when(g >= 1)
                def _():
                    for b in range(4):
                        ostore(g - 1, (s + 2) % _NSLOT, b).wait()

                start_chunk(g + 2, (s + 2) % _NSLOT)

        start_chunk(0, 0)
        start_chunk(1, 1)

        def group(gg, carry):
            for p in range(_NSLOT):
                chunk_work(gg * _NSLOT + p, p)
            return carry

        lax.fori_loop(0, n_groups, group, 0)
        for i in range(tail):
            chunk_work(n_groups * _NSLOT + i, i)

        for g in range(n_chunks - _NSLOT, n_chunks):
            for b in range(4):
                ostore(g, g % _NSLOT, b).wait()

    return body


def kernel(x, emb):
    B, T, D = x.shape
    assert T % (_NW * _S) == 0 and D % 16 == 0
    out = _sc_posenc(B, T, D)(x.reshape(B * T, D), emb)
    return out.reshape(B, T, D)
